# Initial kernel scaffold; baseline (speedup 1.0000x reference)
import jax
import jax.numpy as jnp
from jax.experimental import pallas as pl


def kernel(x, table):
    raise NotImplementedError("write your pallas kernel here")


# speedup vs baseline: 5.3531x; 5.3531x over previous
import functools
import math

import jax
import jax.numpy as jnp
from jax import lax
from jax.experimental import pallas as pl
from jax.experimental.pallas import tpu as pltpu
from jax.experimental.pallas import tpu_sc as plsc

_D = 1024
_MAX_TIMESCALE = 10000.0

_NC = 2
_NS = 16
_NW = _NC * _NS

_CHUNK = 64
_NCHUNK = 4


def _pe_table(seq):
    inc = math.log(_MAX_TIMESCALE) / _D
    inv_timescales = jnp.exp(
        jnp.arange(0, _D, 2, dtype=jnp.float32) * -inc)
    position = jnp.arange(0, seq, dtype=jnp.float32)[:, None]
    pe = jnp.zeros((seq, _D), dtype=jnp.float32)
    pe = pe.at[:, 0::2].set(jnp.sin(position * inv_timescales))
    pe = pe.at[:, 1::2].set(jnp.cos(position * inv_timescales))
    return pe


def _sc_gather(table, idx3):
    n_rows = _NW * _NCHUNK * _CHUNK
    mesh = plsc.VectorSubcoreMesh(core_axis_name="c", subcore_axis_name="s")

    @functools.partial(
        pl.kernel,
        mesh=mesh,
        out_type=jax.ShapeDtypeStruct((n_rows, _D), jnp.float32),
        scratch_types=[
            pltpu.VMEM((_NCHUNK, _CHUNK), jnp.int32),
            pltpu.VMEM((_CHUNK, _D), jnp.float32),
            pltpu.SemaphoreType.DMA,
        ],
    )
    def k(table_hbm, idx_hbm, out_hbm, idx_v, rows_v, sem):
        wid = lax.axis_index("s") * _NC + lax.axis_index("c")
        base = wid * (_NCHUNK * _CHUNK)
        pltpu.sync_copy(idx_hbm.at[wid], idx_v)
        for c in range(_NCHUNK):
            pltpu.async_copy(table_hbm.at[idx_v.at[c]], rows_v, sem).wait()
            pltpu.sync_copy(rows_v, out_hbm.at[pl.ds(base + c * _CHUNK, _CHUNK)])

    return k(table, idx3)


def _fma_body(g_ref, pe_ref, o_ref):
    o_ref[...] = g_ref[...] * math.sqrt(_D) + pe_ref[...]


def kernel(x, table):
    batch, seq = x.shape
    n_rows = batch * seq
    assert n_rows == _NW * _NCHUNK * _CHUNK

    idx3 = x.reshape(_NW, _NCHUNK, _CHUNK)
    g = _sc_gather(table, idx3)

    pe = _pe_table(seq)
    blk = 256
    out = pl.pallas_call(
        _fma_body,
        grid=(n_rows // blk,),
        in_specs=[
            pl.BlockSpec((blk, _D), lambda i: (i, 0)),
            pl.BlockSpec((blk, _D), lambda i: (i % (seq // blk), 0)),
        ],
        out_specs=pl.BlockSpec((blk, _D), lambda i: (i, 0)),
        out_shape=jax.ShapeDtypeStruct((n_rows, _D), jnp.float32),
    )(g, pe)

    return out.reshape(batch, seq, _D)
